# baseline (device time: 18082 ns/iter reference)
import jax
import jax.numpy as jnp
from jax import lax
from jax.experimental import pallas as pl
from jax.experimental.pallas import tpu as pltpu

N_DEV = 4
EPS = 1e-5


def kernel(x, gamma, beta):
    m, n_local = x.shape
    n_global = n_local * N_DEV

    def body(x_ref, g_ref, b_ref, out_ref, comm_ref, send_sems, recv_sems):
        my = lax.axis_index("i")
        peers = [lax.rem(my + d, N_DEV) for d in range(1, N_DEV)]

        barrier_sem = pltpu.get_barrier_semaphore()
        for peer in peers:
            pl.semaphore_signal(
                barrier_sem, inc=1,
                device_id=(peer,), device_id_type=pl.DeviceIdType.MESH,
            )
        pl.semaphore_wait(barrier_sem, N_DEV - 1)

        xf = x_ref[:, :].astype(jnp.float32)
        s1 = jnp.sum(xf, axis=1, keepdims=True)
        s2 = jnp.sum(xf * xf, axis=1, keepdims=True)
        comm_ref[my] = jnp.concatenate([s1, s2], axis=1)

        sends = []
        for d, peer in enumerate(peers):
            rdma = pltpu.make_async_remote_copy(
                src_ref=comm_ref.at[my],
                dst_ref=comm_ref.at[my],
                send_sem=send_sems.at[d],
                recv_sem=recv_sems.at[my],
                device_id=(peer,),
                device_id_type=pl.DeviceIdType.MESH,
            )
            rdma.start()
            sends.append(rdma)

        for d, peer in enumerate(peers):
            recv = pltpu.make_async_remote_copy(
                src_ref=comm_ref.at[peer],
                dst_ref=comm_ref.at[peer],
                send_sem=send_sems.at[d],
                recv_sem=recv_sems.at[peer],
                device_id=(peer,),
                device_id_type=pl.DeviceIdType.MESH,
            )
            recv.wait_recv()
        for rdma in sends:
            rdma.wait_send()

        total = (
            comm_ref[0] + comm_ref[1] + comm_ref[2] + comm_ref[3]
        )
        mean = total[:, 0:1] / n_global
        var = total[:, 1:2] / n_global - mean * mean
        inv = lax.rsqrt(var + EPS)
        g = g_ref[:, :].astype(jnp.float32)
        b = b_ref[:, :].astype(jnp.float32)
        out_ref[:, :] = ((xf - mean) * inv * g + b).astype(out_ref.dtype)

    return pl.pallas_call(
        body,
        out_shape=jax.ShapeDtypeStruct((m, n_local), x.dtype),
        in_specs=[
            pl.BlockSpec(memory_space=pltpu.VMEM),
            pl.BlockSpec(memory_space=pltpu.VMEM),
            pl.BlockSpec(memory_space=pltpu.VMEM),
        ],
        out_specs=pl.BlockSpec(memory_space=pltpu.VMEM),
        scratch_shapes=[
            pltpu.VMEM((N_DEV, m, 2), jnp.float32),
            pltpu.SemaphoreType.DMA((N_DEV - 1,)),
            pltpu.SemaphoreType.DMA((N_DEV,)),
        ],
        compiler_params=pltpu.CompilerParams(collective_id=0),
    )(x, gamma.reshape(1, n_local), beta.reshape(1, n_local))


# device time: 9957 ns/iter; 1.8160x vs baseline; 1.8160x over previous
import jax
import jax.numpy as jnp
from jax import lax
from jax.experimental import pallas as pl
from jax.experimental.pallas import tpu as pltpu

N_DEV = 4
EPS = 1e-5


def kernel(x, gamma, beta):
    m, n_local = x.shape
    n_global = n_local * N_DEV

    def body(x_ref, g_ref, b_ref, out_ref, comm_ref, send_sems, recv_sems):
        my = lax.axis_index("i")
        peers = [lax.rem(my + d, N_DEV) for d in range(1, N_DEV)]

        barrier_sem = pltpu.get_barrier_semaphore()
        for peer in peers:
            pl.semaphore_signal(
                barrier_sem, inc=1,
                device_id=(peer,), device_id_type=pl.DeviceIdType.MESH,
            )
        pl.semaphore_wait(barrier_sem, N_DEV - 1)

        xf = x_ref[:, :].astype(jnp.float32)
        s1 = jnp.sum(xf, axis=1, keepdims=True)
        s2 = jnp.sum(xf * xf, axis=1, keepdims=True)
        comm_ref[my] = jnp.concatenate([s1, s2], axis=1).T

        sends = []
        for d, peer in enumerate(peers):
            rdma = pltpu.make_async_remote_copy(
                src_ref=comm_ref.at[my],
                dst_ref=comm_ref.at[my],
                send_sem=send_sems.at[d],
                recv_sem=recv_sems.at[my],
                device_id=(peer,),
                device_id_type=pl.DeviceIdType.MESH,
            )
            rdma.start()
            sends.append(rdma)

        for d, peer in enumerate(peers):
            recv = pltpu.make_async_remote_copy(
                src_ref=comm_ref.at[peer],
                dst_ref=comm_ref.at[peer],
                send_sem=send_sems.at[d],
                recv_sem=recv_sems.at[peer],
                device_id=(peer,),
                device_id_type=pl.DeviceIdType.MESH,
            )
            recv.wait_recv()
        for rdma in sends:
            rdma.wait_send()

        total = (
            comm_ref[0] + comm_ref[1] + comm_ref[2] + comm_ref[3]
        )
        mean = total[0:1, :].T / n_global
        var = total[1:2, :].T / n_global - mean * mean
        inv = lax.rsqrt(var + EPS)
        g = g_ref[:, :].astype(jnp.float32)
        b = b_ref[:, :].astype(jnp.float32)
        out_ref[:, :] = ((xf - mean) * inv * g + b).astype(out_ref.dtype)

    return pl.pallas_call(
        body,
        out_shape=jax.ShapeDtypeStruct((m, n_local), x.dtype),
        in_specs=[
            pl.BlockSpec(memory_space=pltpu.VMEM),
            pl.BlockSpec(memory_space=pltpu.VMEM),
            pl.BlockSpec(memory_space=pltpu.VMEM),
        ],
        out_specs=pl.BlockSpec(memory_space=pltpu.VMEM),
        scratch_shapes=[
            pltpu.VMEM((N_DEV, 2, m), jnp.float32),
            pltpu.SemaphoreType.DMA((N_DEV - 1,)),
            pltpu.SemaphoreType.DMA((N_DEV,)),
        ],
        compiler_params=pltpu.CompilerParams(collective_id=0),
    )(x, gamma.reshape(1, n_local), beta.reshape(1, n_local))


# device time: 9808 ns/iter; 1.8436x vs baseline; 1.0152x over previous
import jax
import jax.numpy as jnp
from jax import lax
from jax.experimental import pallas as pl
from jax.experimental.pallas import tpu as pltpu

N_DEV = 4
EPS = 1e-5


def kernel(x, gamma, beta):
    m, n_local = x.shape
    n_global = n_local * N_DEV

    def body(x_ref, g_ref, b_ref, out_ref, comm_ref, send_sems, recv_sems):
        my = lax.axis_index("i")
        peers = [lax.rem(my + d, N_DEV) for d in range(1, N_DEV)]

        barrier_sem = pltpu.get_barrier_semaphore()
        for peer in peers:
            pl.semaphore_signal(
                barrier_sem, inc=1,
                device_id=(peer,), device_id_type=pl.DeviceIdType.MESH,
            )
        pl.semaphore_wait(barrier_sem, N_DEV - 1)

        xf = x_ref[:, :].astype(jnp.float32)
        s1 = jnp.sum(xf, axis=1, keepdims=True)
        s2 = jnp.sum(xf * xf, axis=1, keepdims=True)
        comm_ref[my] = jnp.concatenate([s1, s2], axis=1).T

        sends = []
        for d, peer in enumerate(peers):
            rdma = pltpu.make_async_remote_copy(
                src_ref=comm_ref.at[my],
                dst_ref=comm_ref.at[my],
                send_sem=send_sems.at[d],
                recv_sem=recv_sems.at[my],
                device_id=(peer,),
                device_id_type=pl.DeviceIdType.MESH,
            )
            rdma.start()
            sends.append(rdma)

        g = g_ref[:, :].astype(jnp.float32)
        xg = xf * g

        for d, peer in enumerate(peers):
            recv = pltpu.make_async_remote_copy(
                src_ref=comm_ref.at[peer],
                dst_ref=comm_ref.at[peer],
                send_sem=send_sems.at[d],
                recv_sem=recv_sems.at[peer],
                device_id=(peer,),
                device_id_type=pl.DeviceIdType.MESH,
            )
            recv.wait_recv()
        for rdma in sends:
            rdma.wait_send()

        total = (
            comm_ref[0] + comm_ref[1] + comm_ref[2] + comm_ref[3]
        )
        mean = total[0:1, :].T / n_global
        var = total[1:2, :].T / n_global - mean * mean
        inv = lax.rsqrt(var + EPS)
        b = b_ref[:, :].astype(jnp.float32)
        out_ref[:, :] = (xg * inv - (mean * inv) * g + b).astype(out_ref.dtype)

    return pl.pallas_call(
        body,
        out_shape=jax.ShapeDtypeStruct((m, n_local), jnp.bfloat16),
        in_specs=[
            pl.BlockSpec(memory_space=pltpu.VMEM),
            pl.BlockSpec(memory_space=pltpu.VMEM),
            pl.BlockSpec(memory_space=pltpu.VMEM),
        ],
        out_specs=pl.BlockSpec(memory_space=pltpu.VMEM),
        scratch_shapes=[
            pltpu.VMEM((N_DEV, 2, m), jnp.float32),
            pltpu.SemaphoreType.DMA((N_DEV - 1,)),
            pltpu.SemaphoreType.DMA((N_DEV,)),
        ],
        compiler_params=pltpu.CompilerParams(collective_id=0),
    )(x, gamma.reshape(1, n_local), beta.reshape(1, n_local))
